# baseline (speedup 1.0000x reference)
import functools

import jax
import jax.numpy as jnp
from jax import lax
from jax.experimental import pallas as pl
from jax.experimental.pallas import tpu as pltpu
from jax.experimental.pallas import tpu_sc as plsc

NC = 2
NS = 16
L = 16
NW = NC * NS

BATCH = 16384
D = 64
B_PER_W = BATCH // NW
CHUNK = 128
NCHUNK = B_PER_W // CHUNK
GROUPS = CHUNK // L


def _softplus(x):
    ax = jnp.abs(x)
    z = jnp.exp(-ax)
    t = z / (2.0 + z)
    t2 = t * t
    log1p = t * (2.0 + t2 * (2.0 / 3.0 + t2 * (0.4 + t2 * (2.0 / 7.0))))
    return jnp.maximum(x, 0.0) + log1p


def _bpr_body(user_hbm, item_i_hbm, item_j_hbm, eu_hbm, ei_hbm,
              out_sp_hbm, out_sq_hbm,
              uidx, pidx, nidx, ubuf, pbuf, nbuf,
              sp_stage, sq_stage, sem0, sem1):
    wid = lax.axis_index("s") * NC + lax.axis_index("c")
    sems = (sem0, sem1)

    pltpu.sync_copy(user_hbm.at[wid], uidx)
    pltpu.sync_copy(item_i_hbm.at[wid], pidx)
    pltpu.sync_copy(item_j_hbm.at[wid], nidx)

    def fire(c):
        s = c % 2
        pltpu.async_copy(eu_hbm.at[uidx.at[c]], ubuf.at[s], sems[s])
        pltpu.async_copy(ei_hbm.at[pidx.at[c]], pbuf.at[s], sems[s])
        pltpu.async_copy(ei_hbm.at[nidx.at[c]], nbuf.at[s], sems[s])

    def drain(c):
        s = c % 2
        pltpu.make_async_copy(eu_hbm.at[uidx.at[c]], ubuf.at[s], sems[s]).wait()
        pltpu.make_async_copy(ei_hbm.at[pidx.at[c]], pbuf.at[s], sems[s]).wait()
        pltpu.make_async_copy(ei_hbm.at[nidx.at[c]], nbuf.at[s], sems[s]).wait()

    iota = lax.iota(jnp.int32, L)
    zero = jnp.zeros((L,), jnp.float32)

    JBLK = 16

    def compute_chunk(ub, pb, nb, carry):
        def group(g, carry):
            sp_acc, qu0, qu1, qp0, qp1, qn0, qn1 = carry
            row = g * L + iota

            def jblock(jb, inner):
                d = list(inner[0:4])
                q_u = list(inner[4:6])
                q_p = list(inner[6:8])
                q_n = list(inner[8:10])
                base = iota + jb * JBLK
                for jj in range(JBLK):
                    col = jnp.bitwise_and(base + jj, D - 1)
                    u = plsc.load_gather(ub, [row, col])
                    p = plsc.load_gather(pb, [row, col])
                    n = plsc.load_gather(nb, [row, col])
                    d[jj % 4] = d[jj % 4] + u * (n - p)
                    q_u[jj % 2] = q_u[jj % 2] + u * u
                    q_p[jj % 2] = q_p[jj % 2] + p * p
                    q_n[jj % 2] = q_n[jj % 2] + n * n
                return (*d, *q_u, *q_p, *q_n)

            inner = (zero, zero, zero, zero, qu0, qu1, qp0, qp1, qn0, qn1)
            inner = lax.fori_loop(0, D // JBLK, jblock, inner)
            d0, d1, d2, d3, qu0, qu1, qp0, qp1, qn0, qn1 = inner
            dt = (d0 + d1) + (d2 + d3)
            sp_acc = sp_acc + _softplus(dt)
            return (sp_acc, qu0, qu1, qp0, qp1, qn0, qn1)

        return lax.fori_loop(0, GROUPS, group, carry)

    carry = (zero,) * 7
    fire(0)
    for c in range(NCHUNK):
        if c + 1 < NCHUNK:
            fire(c + 1)
        drain(c)
        s = c % 2
        carry = compute_chunk(ubuf.at[s], pbuf.at[s], nbuf.at[s], carry)

    sp_acc, qu0, qu1, qp0, qp1, qn0, qn1 = carry
    sq = (qu0 + qu1) + (qp0 + qp1) + (qn0 + qn1)
    sp_stage[...] = sp_acc
    sq_stage[...] = sq
    pltpu.sync_copy(sp_stage, out_sp_hbm.at[wid])
    pltpu.sync_copy(sq_stage, out_sq_hbm.at[wid])


@jax.jit
def _bpr_call(user_r, item_i_r, item_j_r, embed_user, embed_item):
    mesh = plsc.VectorSubcoreMesh(core_axis_name="c", subcore_axis_name="s")
    f = pl.kernel(
        _bpr_body,
        out_type=[
            jax.ShapeDtypeStruct((NW, L), jnp.float32),
            jax.ShapeDtypeStruct((NW, L), jnp.float32),
        ],
        mesh=mesh,
        compiler_params=pltpu.CompilerParams(
            needs_layout_passes=False, use_tc_tiling_on_sc=False
        ),
        scratch_types=[
            pltpu.VMEM((NCHUNK, CHUNK), jnp.int32),
            pltpu.VMEM((NCHUNK, CHUNK), jnp.int32),
            pltpu.VMEM((NCHUNK, CHUNK), jnp.int32),
            pltpu.VMEM((2, CHUNK, D), jnp.float32),
            pltpu.VMEM((2, CHUNK, D), jnp.float32),
            pltpu.VMEM((2, CHUNK, D), jnp.float32),
            pltpu.VMEM((L,), jnp.float32),
            pltpu.VMEM((L,), jnp.float32),
            pltpu.SemaphoreType.DMA,
            pltpu.SemaphoreType.DMA,
        ],
    )
    sp_part, sq_part = f(user_r, item_i_r, item_j_r, embed_user, embed_item)
    inv_b = 1.0 / BATCH
    loss = jnp.sum(sp_part) * inv_b
    reg = 0.5 * jnp.sum(sq_part) * inv_b
    return loss, reg


def kernel(user, item_i, item_j, embed_user, embed_item):
    user_r = user.astype(jnp.int32).reshape(NW, NCHUNK, CHUNK)
    item_i_r = item_i.astype(jnp.int32).reshape(NW, NCHUNK, CHUNK)
    item_j_r = item_j.astype(jnp.int32).reshape(NW, NCHUNK, CHUNK)
    return _bpr_call(user_r, item_i_r, item_j_r, embed_user, embed_item)
